# Initial kernel scaffold; baseline (speedup 1.0000x reference)
import jax
import jax.numpy as jnp
from jax.experimental import pallas as pl


def kernel(x, edge_index, W, b):
    raise NotImplementedError("write your pallas kernel here")


# speedup vs baseline: 9.1940x; 9.1940x over previous
import functools

import jax
import jax.numpy as jnp
from jax import lax
from jax.experimental import pallas as pl
from jax.experimental.pallas import tpu as pltpu
from jax.experimental.pallas import tpu_sc as plsc

N = 10000
E = 320000
D = 128
NC = 2
NS = 16
CH = 128
NP = 10240
RPT = NP // NS
EPT = 10112
NCHUNK = EPT // CH
EH = NS * EPT
EPP = NC * EH
ZR = 128

_mesh = plsc.VectorSubcoreMesh(core_axis_name="c", subcore_axis_name="s")


@functools.partial(
    pl.kernel,
    out_type=jax.ShapeDtypeStruct((NC, NP), jnp.float32),
    mesh=_mesh,
    scratch_types=[
        pltpu.VMEM((CH,), jnp.int32),
        pltpu.VMEM((CH,), jnp.float32),
        pltpu.VMEM((RPT,), jnp.float32),
        pltpu.VMEM_SHARED((NP,), jnp.float32),
    ],
)
def _deg_call(src_hbm, out_hbm, idx_v, ones_v, zbuf_v, deg_sh):
    cc = lax.axis_index("c")
    ss = lax.axis_index("s")

    def fill(i, _):
        zbuf_v[pl.ds(i * 16, 16)] = jnp.zeros((16,), jnp.float32)
        return 0
    lax.fori_loop(0, RPT // 16, fill, 0)

    def fill1(i, _):
        ones_v[pl.ds(i * 16, 16)] = jnp.ones((16,), jnp.float32)
        return 0
    lax.fori_loop(0, CH // 16, fill1, 0)

    pltpu.sync_copy(zbuf_v, deg_sh.at[pl.ds(ss * RPT, RPT)])
    plsc.subcore_barrier()

    def chunk(j, _):
        base = cc * EH + ss * EPT + j * CH
        pltpu.sync_copy(src_hbm.at[pl.ds(base, CH)], idx_v)
        pltpu.sync_copy(ones_v, deg_sh.at[idx_v], add=True)
        return 0
    lax.fori_loop(0, NCHUNK, chunk, 0)

    plsc.subcore_barrier()
    pltpu.sync_copy(deg_sh.at[pl.ds(ss * RPT, RPT)],
                    out_hbm.at[cc, pl.ds(ss * RPT, RPT)])


_BR = 2048

def _mm_body(x_ref, w_ref, d0_ref, d1_ref, y_ref):
    deg = d0_ref[...] + d1_ref[...]
    dinv = jnp.where(deg > 0.0, 1.0 / deg, 0.0)
    xw = jnp.dot(x_ref[...], w_ref[...], preferred_element_type=jnp.float32)
    y_ref[...] = xw * dinv


_mm_call = pl.pallas_call(
    _mm_body,
    grid=(NP // _BR,),
    in_specs=[
        pl.BlockSpec((_BR, D), lambda i: (i, 0)),
        pl.BlockSpec((D, D), lambda i: (0, 0)),
        pl.BlockSpec((_BR, 1), lambda i: (i, 0)),
        pl.BlockSpec((_BR, 1), lambda i: (i, 0)),
    ],
    out_specs=pl.BlockSpec((_BR, D), lambda i: (i, 0)),
    out_shape=jax.ShapeDtypeStruct((NP, D), jnp.float32),
)


@functools.partial(
    pl.kernel,
    out_type=jax.ShapeDtypeStruct((NC, NP, D), jnp.float32),
    mesh=_mesh,
    scratch_types=[
        pltpu.VMEM((CH,), jnp.int32),
        pltpu.VMEM((CH,), jnp.int32),
        pltpu.VMEM((CH, D), jnp.float32),
        pltpu.VMEM((ZR, D), jnp.float32),
        pltpu.VMEM_SHARED((NP, D), jnp.float32),
    ],
)
def _gs_call(y_hbm, src_hbm, dst_hbm, out_hbm, sidx_v, didx_v, rows_v,
             zbuf_v, acc_sh):
    cc = lax.axis_index("c")
    ss = lax.axis_index("s")

    def fill(i, _):
        r = i // (D // 16)
        c = i % (D // 16)
        zbuf_v[r, pl.ds(c * 16, 16)] = jnp.zeros((16,), jnp.float32)
        return 0
    lax.fori_loop(0, ZR * (D // 16), fill, 0)

    def zcopy(k, _):
        pltpu.sync_copy(zbuf_v, acc_sh.at[pl.ds(ss * RPT + k * ZR, ZR)])
        return 0
    lax.fori_loop(0, RPT // ZR, zcopy, 0)
    plsc.subcore_barrier()

    def chunk(j, _):
        base = cc * EH + ss * EPT + j * CH
        pltpu.sync_copy(src_hbm.at[pl.ds(base, CH)], sidx_v)
        pltpu.sync_copy(dst_hbm.at[pl.ds(base, CH)], didx_v)
        pltpu.sync_copy(y_hbm.at[sidx_v], rows_v)
        pltpu.sync_copy(rows_v, acc_sh.at[didx_v], add=True)
        return 0
    lax.fori_loop(0, NCHUNK, chunk, 0)

    plsc.subcore_barrier()
    pltpu.sync_copy(acc_sh.at[pl.ds(ss * RPT, RPT)],
                    out_hbm.at[cc, pl.ds(ss * RPT, RPT)])


_BO = 2000

def _comb_body(p_ref, b_ref, o_ref):
    o_ref[...] = p_ref[0] + p_ref[1] + b_ref[...]


_comb_call = pl.pallas_call(
    _comb_body,
    grid=(N // _BO,),
    in_specs=[
        pl.BlockSpec((NC, _BO, D), lambda i: (0, i, 0)),
        pl.BlockSpec((1, D), lambda i: (0, 0)),
    ],
    out_specs=pl.BlockSpec((_BO, D), lambda i: (i, 0)),
    out_shape=jax.ShapeDtypeStruct((N, D), jnp.float32),
)


def kernel(x, edge_index, W, b):
    src = edge_index[0].astype(jnp.int32)
    dst = edge_index[1].astype(jnp.int32)
    pad = jnp.full((EPP - E,), N, dtype=jnp.int32)
    src_p = jnp.concatenate([src, pad])
    dst_p = jnp.concatenate([dst, pad])
    x_p = jnp.concatenate([x, jnp.zeros((NP - N, D), x.dtype)])

    degs = _deg_call(src_p)
    d0 = degs[0].reshape(NP, 1)
    d1 = degs[1].reshape(NP, 1)
    y = _mm_call(x_p, W, d0, d1)
    parts = _gs_call(y, src_p, dst_p)
    return _comb_call(parts, b.reshape(1, D))
